# baseline (speedup 1.0000x reference)
import jax
import jax.numpy as jnp
from jax import lax
from jax.experimental import pallas as pl
from jax.experimental.pallas import tpu as pltpu

VOCAB = 100000
OBJ = 100000
DIM = 200
N_NEG = 100
N_IDX = N_NEG + 1
LANE = 128
NBUF = 8


def _body(cols, word, o_any, w_any, res, wout, w_vmem, sem):
    wi = word[0]
    cp = pltpu.make_async_copy(
        w_any.at[pl.ds(pl.multiple_of((wi // 8) * 8, 8), 8), :], w_vmem, sem)
    cp.start()
    cp.wait()
    w = w_vmem[pl.ds(lax.rem(wi, 8), 1), :]
    wsq = jnp.sum(w * w)
    wout[...] = w
    res[...] = jnp.zeros((1, LANE), jnp.float32)
    lane_iota = lax.broadcasted_iota(jnp.int32, (1, LANE), 1)
    eps2 = jnp.float32(1e-16)

    def step(idx, *o_blks):
        i = idx[0]
        lane = lax.rem(cols[i], LANE)
        o = jnp.concatenate([b[...] for b in o_blks], axis=0)
        num_full = jnp.dot(w, o, preferred_element_type=jnp.float32,
                           precision=lax.Precision.HIGHEST)
        sq_full = jnp.sum(o * o, axis=0, keepdims=True)
        denom2 = jnp.maximum(wsq, eps2) * jnp.maximum(sq_full, eps2)
        r_vec = num_full * lax.rsqrt(denom2)
        r_scalar = jnp.sum(jnp.where(lane_iota == lane, r_vec, 0.0))
        res[...] = jnp.where(lane_iota == i, r_scalar, res[...])

    pltpu.emit_pipeline(
        step,
        grid=(N_IDX,),
        in_specs=[
            pl.BlockSpec((8, LANE),
                         lambda i, r=r: (r, cols[i] // LANE),
                         pipeline_mode=pl.Buffered(buffer_count=4))
            for r in range(DIM // 8)
        ],
        _explicit_indices=True,
    )(*([o_any] * (DIM // 8)))


_tc_call = pl.pallas_call(
    _body,
    in_specs=[
        pl.BlockSpec(memory_space=pltpu.SMEM),
        pl.BlockSpec(memory_space=pltpu.SMEM),
        pl.BlockSpec(memory_space=pl.ANY),
        pl.BlockSpec(memory_space=pl.ANY),
    ],
    out_specs=[
        pl.BlockSpec(memory_space=pltpu.VMEM),
        pl.BlockSpec(memory_space=pltpu.VMEM),
    ],
    out_shape=(
        jax.ShapeDtypeStruct((1, LANE), jnp.float32),
        jax.ShapeDtypeStruct((1, DIM), jnp.float32),
    ),
    scratch_shapes=[
        pltpu.VMEM((8, DIM), jnp.float32),
        pltpu.SemaphoreType.DMA,
    ],
)


def kernel(W, O, word, obj, neg_samples):
    word = jnp.asarray(word, jnp.int32).reshape(1)
    obj = jnp.asarray(obj, jnp.int32)
    neg = jnp.asarray(neg_samples, jnp.int32)
    cols = jnp.concatenate([obj.reshape(1), neg])

    res, wout = _tc_call(cols, word, O, W)
    word_embed = wout
    obj_embed = res[0, 0]
    neg_embeds = res[0, 1:1 + N_NEG]
    return (word_embed, obj_embed, neg_embeds)


# speedup vs baseline: 1.6055x; 1.6055x over previous
import jax
import jax.numpy as jnp
from jax import lax
from jax.experimental import pallas as pl
from jax.experimental.pallas import tpu as pltpu

VOCAB = 100000
OBJ = 100000
DIM = 200
N_NEG = 100
N_IDX = N_NEG + 1
LANE = 128
NBUF = 16


def _body(cols, word, o_any, w_any, res, wout, w_vmem, sem):
    wi = word[0]
    cp = pltpu.make_async_copy(
        w_any.at[pl.ds(pl.multiple_of((wi // 8) * 8, 8), 8), :], w_vmem, sem)
    cp.start()
    cp.wait()
    w = w_vmem[pl.ds(lax.rem(wi, 8), 1), :]
    wsq = jnp.sum(w * w)
    wout[...] = w
    res[...] = jnp.zeros((1, LANE), jnp.float32)
    lane_iota = lax.broadcasted_iota(jnp.int32, (1, LANE), 1)
    eps2 = jnp.float32(1e-16)

    def step(idx, o_blk):
        i = idx[0]
        lane = lax.rem(cols[i], LANE)
        o = o_blk[...]
        num_full = jnp.dot(w, o, preferred_element_type=jnp.float32,
                           precision=lax.Precision.HIGHEST)
        sq_full = jnp.sum(o * o, axis=0, keepdims=True)
        denom2 = jnp.maximum(wsq, eps2) * jnp.maximum(sq_full, eps2)
        r_vec = num_full * lax.rsqrt(denom2)
        r_scalar = jnp.sum(jnp.where(lane_iota == lane, r_vec, 0.0))
        res[...] = jnp.where(lane_iota == i, r_scalar, res[...])

    pltpu.emit_pipeline(
        step,
        grid=(N_IDX,),
        in_specs=[pl.BlockSpec((DIM, LANE), lambda i: (0, cols[i] // LANE),
                               pipeline_mode=pl.Buffered(buffer_count=NBUF))],
        _explicit_indices=True,
    )(o_any)


_tc_call = pl.pallas_call(
    _body,
    in_specs=[
        pl.BlockSpec(memory_space=pltpu.SMEM),
        pl.BlockSpec(memory_space=pltpu.SMEM),
        pl.BlockSpec(memory_space=pl.ANY),
        pl.BlockSpec(memory_space=pl.ANY),
    ],
    out_specs=[
        pl.BlockSpec(memory_space=pltpu.VMEM),
        pl.BlockSpec(memory_space=pltpu.VMEM),
    ],
    out_shape=(
        jax.ShapeDtypeStruct((1, LANE), jnp.float32),
        jax.ShapeDtypeStruct((1, DIM), jnp.float32),
    ),
    scratch_shapes=[
        pltpu.VMEM((8, DIM), jnp.float32),
        pltpu.SemaphoreType.DMA,
    ],
)


def kernel(W, O, word, obj, neg_samples):
    word = jnp.asarray(word, jnp.int32).reshape(1)
    obj = jnp.asarray(obj, jnp.int32)
    neg = jnp.asarray(neg_samples, jnp.int32)
    cols = jnp.concatenate([obj.reshape(1), neg])

    res, wout = _tc_call(cols, word, O, W)
    word_embed = wout
    obj_embed = res[0, 0]
    neg_embeds = res[0, 1:1 + N_NEG]
    return (word_embed, obj_embed, neg_embeds)


# speedup vs baseline: 1.6356x; 1.0187x over previous
import jax
import jax.numpy as jnp
from jax import lax
from jax.experimental import pallas as pl
from jax.experimental.pallas import tpu as pltpu

VOCAB = 100000
OBJ = 100000
DIM = 200
N_NEG = 100
N_IDX = N_NEG + 1
LANE = 128
NBUF = 8


def _body(cols, word, o_any, w_any, res, wout, w_vmem, sem):
    wi = word[0]
    cp = pltpu.make_async_copy(
        w_any.at[pl.ds(pl.multiple_of((wi // 8) * 8, 8), 8), :], w_vmem, sem)
    cp.start()
    cp.wait()
    w = w_vmem[pl.ds(lax.rem(wi, 8), 1), :]
    wsq = jnp.sum(w * w)
    wout[...] = w
    res[...] = jnp.zeros((1, LANE), jnp.float32)
    lane_iota = lax.broadcasted_iota(jnp.int32, (1, LANE), 1)
    eps2 = jnp.float32(1e-16)

    def step(idx, o_blk):
        i = idx[0]
        lane = lax.rem(cols[i], LANE)
        o = o_blk[...]
        num_full = jnp.dot(w, o, preferred_element_type=jnp.float32,
                           precision=lax.Precision.HIGHEST)
        sq_full = jnp.sum(o * o, axis=0, keepdims=True)
        denom2 = jnp.maximum(wsq, eps2) * jnp.maximum(sq_full, eps2)
        r_vec = num_full * lax.rsqrt(denom2)
        r_scalar = jnp.sum(jnp.where(lane_iota == lane, r_vec, 0.0))
        res[...] = jnp.where(lane_iota == i, r_scalar, res[...])

    pltpu.emit_pipeline(
        step,
        grid=(N_IDX,),
        in_specs=[pl.BlockSpec((DIM, LANE), lambda i: (0, cols[i] // LANE),
                               pipeline_mode=pl.Buffered(buffer_count=NBUF))],
        _explicit_indices=True,
    )(o_any)


_tc_call = pl.pallas_call(
    _body,
    in_specs=[
        pl.BlockSpec(memory_space=pltpu.SMEM),
        pl.BlockSpec(memory_space=pltpu.SMEM),
        pl.BlockSpec(memory_space=pl.ANY),
        pl.BlockSpec(memory_space=pl.ANY),
    ],
    out_specs=[
        pl.BlockSpec(memory_space=pltpu.VMEM),
        pl.BlockSpec(memory_space=pltpu.VMEM),
    ],
    out_shape=(
        jax.ShapeDtypeStruct((1, LANE), jnp.float32),
        jax.ShapeDtypeStruct((1, DIM), jnp.float32),
    ),
    scratch_shapes=[
        pltpu.VMEM((8, DIM), jnp.float32),
        pltpu.SemaphoreType.DMA,
    ],
)


def kernel(W, O, word, obj, neg_samples):
    word = jnp.asarray(word, jnp.int32).reshape(1)
    obj = jnp.asarray(obj, jnp.int32)
    neg = jnp.asarray(neg_samples, jnp.int32)
    cols = jnp.concatenate([obj.reshape(1), neg])

    res, wout = _tc_call(cols, word, O, W)
    word_embed = wout
    obj_embed = res[0, 0]
    neg_embeds = res[0, 1:1 + N_NEG]
    return (word_embed, obj_embed, neg_embeds)
